# baseline (speedup 1.0000x reference)
import functools

import jax
import jax.numpy as jnp
from jax import lax
from jax.experimental import pallas as pl
from jax.experimental.pallas import tpu as pltpu
from jax.experimental.pallas import tpu_sc as plsc

_NC = 2
_NS = 16
_CHUNK = 128
_DW = 8


def _make_segsum(acc_rows, w, rows_per_tile, nbuf, with_deg):
    mesh = plsc.VectorSubcoreMesh(core_axis_name="c", subcore_axis_name="s")
    zrows = acc_rows // _NS

    out_type = [jax.ShapeDtypeStruct((_NC, acc_rows, w), jnp.float32)]
    scratch = [
        pltpu.VMEM((rows_per_tile, _CHUNK), jnp.int32),
        pltpu.VMEM((rows_per_tile, _CHUNK), jnp.int32),
        [pltpu.VMEM((_CHUNK, w), jnp.float32) for _ in range(nbuf)],
        [pltpu.SemaphoreType.DMA for _ in range(nbuf)],
        pltpu.VMEM_SHARED((acc_rows, w), jnp.float32),
        pltpu.VMEM_SHARED((acc_rows, w), jnp.float32),
    ]
    if with_deg:
        out_type.append(jax.ShapeDtypeStruct((_NC, acc_rows, _DW), jnp.float32))
        scratch.append(pltpu.VMEM((_CHUNK, _DW), jnp.float32))
        scratch.append(pltpu.VMEM_SHARED((acc_rows, _DW), jnp.float32))

    @functools.partial(
        pl.kernel,
        out_type=out_type,
        mesh=mesh,
        scratch_types=scratch,
        compiler_params=pltpu.CompilerParams(use_tc_tiling_on_sc=False),
    )
    def segsum(table_hbm, src_hbm, dst_hbm, zeros_hbm, *rest):
        if with_deg:
            (onesd_hbm, out_hbm, deg_hbm,
             src_v, dst_v, bufs, sems, table_sh, acc_sh,
             ones_v, deg_sh) = rest
        else:
            out_hbm, src_v, dst_v, bufs, sems, table_sh, acc_sh = rest
        c = lax.axis_index("c")
        s = lax.axis_index("s")
        wid = c * _NS + s
        sl = pl.ds(s * zrows, zrows)

        base = wid * rows_per_tile
        stage = [
            pltpu.async_copy(table_hbm.at[sl], table_sh.at[sl], sems[0]),
            pltpu.async_copy(zeros_hbm.at[sl, pl.ds(0, w)], acc_sh.at[sl],
                             sems[0]),
            pltpu.async_copy(src_hbm.at[pl.ds(base, rows_per_tile)], src_v,
                             sems[0]),
            pltpu.async_copy(dst_hbm.at[pl.ds(base, rows_per_tile)], dst_v,
                             sems[0]),
        ]
        if with_deg:
            stage.append(pltpu.async_copy(
                zeros_hbm.at[sl, pl.ds(0, _DW)], deg_sh.at[sl], sems[0]))
            stage.append(pltpu.async_copy(onesd_hbm, ones_v, sems[0]))
        for cp in stage:
            cp.wait()
        plsc.subcore_barrier()

        last = rows_per_tile - 1
        for b in range(nbuf):
            pltpu.async_copy(table_sh.at[src_v.at[min(b, last)]],
                             bufs[b], sems[b])

        def body(g, carry):
            for b in range(nbuf):
                j = g * nbuf + b
                pltpu.make_async_copy(
                    table_sh.at[src_v.at[j]], bufs[b], sems[b]).wait()
                pltpu.sync_copy(bufs[b], acc_sh.at[dst_v.at[j]], add=True)
                pltpu.async_copy(
                    table_sh.at[src_v.at[jnp.minimum(j + nbuf, last)]],
                    bufs[b], sems[b])
                if with_deg:
                    pltpu.sync_copy(ones_v, deg_sh.at[dst_v.at[j]], add=True)
            return carry

        lax.fori_loop(0, rows_per_tile // nbuf, body, 0)
        for b in range(nbuf):
            pltpu.make_async_copy(
                table_sh.at[src_v.at[last]], bufs[b], sems[b]).wait()
        plsc.subcore_barrier()

        pub = [pltpu.async_copy(acc_sh.at[sl], out_hbm.at[c, sl], sems[0])]
        if with_deg:
            pub.append(pltpu.async_copy(deg_sh.at[sl], deg_hbm.at[c, sl],
                                        sems[0]))
        for cp in pub:
            cp.wait()

    return segsum


def _tc1_body(x_ref, wl_ref, wr_ref, b_ref, y_ref, r_ref):
    xb = x_ref[...]
    y_ref[...] = jnp.dot(xb, wl_ref[...], preferred_element_type=jnp.float32)
    r_ref[...] = jnp.dot(xb, wr_ref[...], preferred_element_type=jnp.float32) + b_ref[...]


def _tc2_body(p_ref, dg_ref, r1_ref, wl_ref, wr_ref, b_ref, y2_ref, r2_ref):
    agg = p_ref[0] + p_ref[1]
    deg = dg_ref[0, :, 0:1] + dg_ref[1, :, 0:1]
    recip = 1.0 / jnp.maximum(deg, 1.0)
    h = jnp.maximum(agg * recip + r1_ref[...], 0.0)
    y2_ref[...] = jnp.dot(h, wl_ref[...], preferred_element_type=jnp.float32)
    r2_ref[...] = jnp.dot(h, wr_ref[...], preferred_element_type=jnp.float32) + b_ref[...]


def _tc3_body(p2_ref, dg_ref, r2_ref, out_ref):
    ssum = p2_ref[0] + p2_ref[1]
    deg = dg_ref[0, :, 0:1] + dg_ref[1, :, 0:1]
    recip = 1.0 / jnp.maximum(deg, 1.0)
    out_ref[...] = ssum * recip + r2_ref[...]


def kernel(x, edge_index, W1l, b1, W1r, W2l, b2, W2r):
    n, d = x.shape
    h = W1l.shape[1]
    e = edge_index.shape[1]

    rows_per_tile = -(-e // (_NC * _NS * _CHUNK * 8)) * 8
    chunks = rows_per_tile * _NC * _NS
    e_pad = chunks * _CHUNK
    src = edge_index[0].astype(jnp.int32)
    dst = edge_index[1].astype(jnp.int32)
    src_p = jnp.concatenate([src, jnp.zeros((e_pad - e,), jnp.int32)])
    dst_p = jnp.concatenate([dst, jnp.full((e_pad - e,), n, jnp.int32)])
    src2d = src_p.reshape(chunks, _CHUNK)
    dst2d = dst_p.reshape(chunks, _CHUNK)

    acc_rows = -(-(n + 1) // (_NS * 8)) * (_NS * 8)
    x_p = jnp.pad(x, ((0, acc_rows - n), (0, 0)))
    zeros64 = jnp.zeros((acc_rows, h), jnp.float32)
    ones8 = jnp.ones((_CHUNK, _DW), jnp.float32)

    b1r = b1.reshape(1, h)
    b2r = b2.reshape(1, h)

    blk = acc_rows // _NS
    ngrid = _NS

    y1, r1 = pl.pallas_call(
        _tc1_body,
        grid=(ngrid,),
        in_specs=[
            pl.BlockSpec((blk, d), lambda i: (i, 0)),
            pl.BlockSpec((d, h), lambda i: (0, 0)),
            pl.BlockSpec((d, h), lambda i: (0, 0)),
            pl.BlockSpec((1, h), lambda i: (0, 0)),
        ],
        out_specs=[
            pl.BlockSpec((blk, h), lambda i: (i, 0)),
            pl.BlockSpec((blk, h), lambda i: (i, 0)),
        ],
        out_shape=[
            jax.ShapeDtypeStruct((acc_rows, h), jnp.float32),
            jax.ShapeDtypeStruct((acc_rows, h), jnp.float32),
        ],
    )(x_p, W1l, W1r, b1r)

    p1, dg = _make_segsum(acc_rows, h, rows_per_tile, 2, True)(
        y1, src2d, dst2d, zeros64, ones8)

    y2, r2 = pl.pallas_call(
        _tc2_body,
        grid=(ngrid,),
        in_specs=[
            pl.BlockSpec((_NC, blk, h), lambda i: (0, i, 0)),
            pl.BlockSpec((_NC, blk, _DW), lambda i: (0, i, 0)),
            pl.BlockSpec((blk, h), lambda i: (i, 0)),
            pl.BlockSpec((h, h), lambda i: (0, 0)),
            pl.BlockSpec((h, h), lambda i: (0, 0)),
            pl.BlockSpec((1, h), lambda i: (0, 0)),
        ],
        out_specs=[
            pl.BlockSpec((blk, h), lambda i: (i, 0)),
            pl.BlockSpec((blk, h), lambda i: (i, 0)),
        ],
        out_shape=[
            jax.ShapeDtypeStruct((acc_rows, h), jnp.float32),
            jax.ShapeDtypeStruct((acc_rows, h), jnp.float32),
        ],
    )(p1, dg, r1, W2l, W2r, b2r)

    (p2,) = _make_segsum(acc_rows, h, rows_per_tile, 2, False)(
        y2, src2d, dst2d, zeros64)

    oblk = 1000 if n % 1000 == 0 else n
    out = pl.pallas_call(
        _tc3_body,
        grid=(n // oblk,),
        in_specs=[
            pl.BlockSpec((_NC, oblk, h), lambda i: (0, i, 0)),
            pl.BlockSpec((_NC, oblk, _DW), lambda i: (0, i, 0)),
            pl.BlockSpec((oblk, h), lambda i: (i, 0)),
        ],
        out_specs=pl.BlockSpec((oblk, h), lambda i: (i, 0)),
        out_shape=jax.ShapeDtypeStruct((n, h), jnp.float32),
    )(p2, dg, r2)

    return out


# speedup vs baseline: 1.0279x; 1.0279x over previous
import functools

import jax
import jax.numpy as jnp
from jax import lax
from jax.experimental import pallas as pl
from jax.experimental.pallas import tpu as pltpu
from jax.experimental.pallas import tpu_sc as plsc

_NC = 2
_NS = 16
_CHUNK = 128
_DW = 8


def _make_segsum(acc_rows, w, rows_per_tile, nbuf, with_deg):
    mesh = plsc.VectorSubcoreMesh(core_axis_name="c", subcore_axis_name="s")
    zrows = acc_rows // _NS

    out_type = [jax.ShapeDtypeStruct((_NC, acc_rows, w), jnp.float32)]
    scratch = [
        pltpu.VMEM((rows_per_tile, _CHUNK), jnp.int32),
        pltpu.VMEM((rows_per_tile, _CHUNK), jnp.int32),
        [pltpu.VMEM((_CHUNK, w), jnp.float32) for _ in range(nbuf)],
        [pltpu.SemaphoreType.DMA for _ in range(nbuf)],
        pltpu.VMEM_SHARED((acc_rows, w), jnp.float32),
        pltpu.VMEM_SHARED((acc_rows, w), jnp.float32),
    ]
    if with_deg:
        out_type.append(jax.ShapeDtypeStruct((_NC, acc_rows, _DW), jnp.float32))
        scratch.append(pltpu.VMEM((_CHUNK, _DW), jnp.float32))
        scratch.append(pltpu.VMEM_SHARED((acc_rows, _DW), jnp.float32))

    @functools.partial(
        pl.kernel,
        out_type=out_type,
        mesh=mesh,
        scratch_types=scratch,
        compiler_params=pltpu.CompilerParams(use_tc_tiling_on_sc=False),
    )
    def segsum(table_hbm, src_hbm, dst_hbm, zeros_hbm, *rest):
        if with_deg:
            (onesd_hbm, out_hbm, deg_hbm,
             src_v, dst_v, bufs, sems, table_sh, acc_sh,
             ones_v, deg_sh) = rest
        else:
            out_hbm, src_v, dst_v, bufs, sems, table_sh, acc_sh = rest
        c = lax.axis_index("c")
        s = lax.axis_index("s")
        wid = c * _NS + s
        sl = pl.ds(s * zrows, zrows)

        base = wid * rows_per_tile
        stage = [
            pltpu.async_copy(table_hbm.at[sl], table_sh.at[sl], sems[0]),
            pltpu.async_copy(zeros_hbm.at[sl, pl.ds(0, w)], acc_sh.at[sl],
                             sems[0]),
            pltpu.async_copy(src_hbm.at[pl.ds(base, rows_per_tile)], src_v,
                             sems[0]),
            pltpu.async_copy(dst_hbm.at[pl.ds(base, rows_per_tile)], dst_v,
                             sems[0]),
        ]
        if with_deg:
            stage.append(pltpu.async_copy(
                zeros_hbm.at[sl, pl.ds(0, _DW)], deg_sh.at[sl], sems[0]))
            stage.append(pltpu.async_copy(onesd_hbm, ones_v, sems[0]))
        for cp in stage:
            cp.wait()
        plsc.subcore_barrier()

        last = rows_per_tile - 1
        for b in range(nbuf):
            pltpu.async_copy(table_sh.at[src_v.at[min(b, last)]],
                             bufs[b], sems[b])

        def body(g, carry):
            for b in range(nbuf):
                j = g * nbuf + b
                pltpu.make_async_copy(
                    table_sh.at[src_v.at[j]], bufs[b], sems[b]).wait()
                pltpu.sync_copy(bufs[b], acc_sh.at[dst_v.at[j]], add=True)
                pltpu.async_copy(
                    table_sh.at[src_v.at[jnp.minimum(j + nbuf, last)]],
                    bufs[b], sems[b])
                if with_deg:
                    pltpu.sync_copy(ones_v, deg_sh.at[dst_v.at[j]], add=True)
            return carry

        lax.fori_loop(0, rows_per_tile // nbuf, body, 0)
        for b in range(nbuf):
            pltpu.make_async_copy(
                table_sh.at[src_v.at[last]], bufs[b], sems[b]).wait()
        plsc.subcore_barrier()

        pub = [pltpu.async_copy(acc_sh.at[sl], out_hbm.at[c, sl], sems[0])]
        if with_deg:
            pub.append(pltpu.async_copy(deg_sh.at[sl], deg_hbm.at[c, sl],
                                        sems[0]))
        for cp in pub:
            cp.wait()

    return segsum


def _proj_body(x_ref, w_ref, b_ref, o_ref):
    o_ref[...] = jnp.dot(x_ref[...], w_ref[...],
                         preferred_element_type=jnp.float32) + b_ref[...]


def _tc2_body(p_ref, dg_ref, r1_ref, wl_ref, h_ref, y2_ref):
    agg = p_ref[0] + p_ref[1]
    deg = dg_ref[0, :, 0:1] + dg_ref[1, :, 0:1]
    recip = 1.0 / jnp.maximum(deg, 1.0)
    h = jnp.maximum(agg * recip + r1_ref[...], 0.0)
    h_ref[...] = h
    y2_ref[...] = jnp.dot(h, wl_ref[...], preferred_element_type=jnp.float32)


def _tc3_body(p2_ref, dg_ref, r2_ref, out_ref):
    ssum = p2_ref[0] + p2_ref[1]
    deg = dg_ref[0, :, 0:1] + dg_ref[1, :, 0:1]
    recip = 1.0 / jnp.maximum(deg, 1.0)
    out_ref[...] = ssum * recip + r2_ref[...]


def kernel(x, edge_index, W1l, b1, W1r, W2l, b2, W2r):
    n, d = x.shape
    h = W1l.shape[1]
    e = edge_index.shape[1]

    rows_per_tile = -(-e // (_NC * _NS * _CHUNK * 8)) * 8
    chunks = rows_per_tile * _NC * _NS
    e_pad = chunks * _CHUNK
    src = edge_index[0].astype(jnp.int32)
    dst = edge_index[1].astype(jnp.int32)
    src_p = jnp.concatenate([src, jnp.zeros((e_pad - e,), jnp.int32)])
    dst_p = jnp.concatenate([dst, jnp.full((e_pad - e,), n, jnp.int32)])
    src2d = src_p.reshape(chunks, _CHUNK)
    dst2d = dst_p.reshape(chunks, _CHUNK)

    acc_rows = -(-(n + 1) // (_NS * 8)) * (_NS * 8)
    zeros64 = jnp.zeros((acc_rows, h), jnp.float32)
    ones8 = jnp.ones((_CHUNK, _DW), jnp.float32)

    b1r = b1.reshape(1, h)
    b2r = b2.reshape(1, h)
    zb = jnp.zeros((1, h), jnp.float32)

    xblk = 1000 if n % 1000 == 0 else n
    xgrid = n // xblk
    blk = acc_rows // _NS
    ngrid = _NS

    def _proj(inp, w, b, rows, iblk, igrid, wdim):
        return pl.pallas_call(
            _proj_body,
            grid=(igrid,),
            in_specs=[
                pl.BlockSpec((iblk, wdim), lambda i: (i, 0)),
                pl.BlockSpec((wdim, h), lambda i: (0, 0)),
                pl.BlockSpec((1, h), lambda i: (0, 0)),
            ],
            out_specs=pl.BlockSpec((iblk, h), lambda i: (i, 0)),
            out_shape=jax.ShapeDtypeStruct((rows, h), jnp.float32),
        )(inp, w, b)

    y1 = _proj(x, W1l, zb, acc_rows, xblk, xgrid, d)
    r1 = _proj(x, W1r, b1r, acc_rows, xblk, xgrid, d)

    p1, dg = _make_segsum(acc_rows, h, rows_per_tile, 2, True)(
        y1, src2d, dst2d, zeros64, ones8)

    harr, y2 = pl.pallas_call(
        _tc2_body,
        grid=(ngrid,),
        in_specs=[
            pl.BlockSpec((_NC, blk, h), lambda i: (0, i, 0)),
            pl.BlockSpec((_NC, blk, _DW), lambda i: (0, i, 0)),
            pl.BlockSpec((blk, h), lambda i: (i, 0)),
            pl.BlockSpec((h, h), lambda i: (0, 0)),
        ],
        out_specs=[
            pl.BlockSpec((blk, h), lambda i: (i, 0)),
            pl.BlockSpec((blk, h), lambda i: (i, 0)),
        ],
        out_shape=[
            jax.ShapeDtypeStruct((acc_rows, h), jnp.float32),
            jax.ShapeDtypeStruct((acc_rows, h), jnp.float32),
        ],
    )(p1, dg, r1, W2l)

    (p2,) = _make_segsum(acc_rows, h, rows_per_tile, 2, False)(
        y2, src2d, dst2d, zeros64)
    r2 = _proj(harr, W2r, b2r, acc_rows, blk, ngrid, h)

    oblk = 1000 if n % 1000 == 0 else n
    out = pl.pallas_call(
        _tc3_body,
        grid=(n // oblk,),
        in_specs=[
            pl.BlockSpec((_NC, oblk, h), lambda i: (0, i, 0)),
            pl.BlockSpec((_NC, oblk, _DW), lambda i: (0, i, 0)),
            pl.BlockSpec((oblk, h), lambda i: (i, 0)),
        ],
        out_specs=pl.BlockSpec((oblk, h), lambda i: (i, 0)),
        out_shape=jax.ShapeDtypeStruct((n, h), jnp.float32),
    )(p2, dg, r2)

    return out


# speedup vs baseline: 1.0618x; 1.0330x over previous
import functools

import jax
import jax.numpy as jnp
from jax import lax
from jax.experimental import pallas as pl
from jax.experimental.pallas import tpu as pltpu
from jax.experimental.pallas import tpu_sc as plsc

_NC = 2
_NS = 16
_CHUNK = 128
_W1 = 72
_DW = 8


def _make_segsum(acc_rows, w, rows_per_tile, nbuf):
    mesh = plsc.VectorSubcoreMesh(core_axis_name="c", subcore_axis_name="s")
    zrows = acc_rows // _NS

    @functools.partial(
        pl.kernel,
        out_type=jax.ShapeDtypeStruct((_NC, acc_rows, w), jnp.float32),
        mesh=mesh,
        scratch_types=[
            pltpu.VMEM((rows_per_tile, _CHUNK), jnp.int32),
            pltpu.VMEM((rows_per_tile, _CHUNK), jnp.int32),
            [pltpu.VMEM((_CHUNK, w), jnp.float32) for _ in range(nbuf)],
            [pltpu.SemaphoreType.DMA for _ in range(nbuf)],
            pltpu.VMEM_SHARED((acc_rows, w), jnp.float32),
            pltpu.VMEM_SHARED((acc_rows, w), jnp.float32),
        ],
        compiler_params=pltpu.CompilerParams(use_tc_tiling_on_sc=False),
    )
    def segsum(table_hbm, src_hbm, dst_hbm, zeros_hbm, out_hbm,
               src_v, dst_v, bufs, sems, table_sh, acc_sh):
        c = lax.axis_index("c")
        s = lax.axis_index("s")
        wid = c * _NS + s
        sl = pl.ds(s * zrows, zrows)

        base = wid * rows_per_tile
        stage = [
            pltpu.async_copy(table_hbm.at[sl], table_sh.at[sl], sems[0]),
            pltpu.async_copy(zeros_hbm.at[sl, pl.ds(0, w)], acc_sh.at[sl],
                             sems[0]),
            pltpu.async_copy(src_hbm.at[pl.ds(base, rows_per_tile)], src_v,
                             sems[0]),
            pltpu.async_copy(dst_hbm.at[pl.ds(base, rows_per_tile)], dst_v,
                             sems[0]),
        ]
        for cp in stage:
            cp.wait()
        plsc.subcore_barrier()

        last = rows_per_tile - 1
        for b in range(nbuf):
            pltpu.async_copy(table_sh.at[src_v.at[min(b, last)]],
                             bufs[b], sems[b])

        def body(g, carry):
            for b in range(nbuf):
                j = g * nbuf + b
                pltpu.make_async_copy(
                    table_sh.at[src_v.at[j]], bufs[b], sems[b]).wait()
                pltpu.sync_copy(bufs[b], acc_sh.at[dst_v.at[j]], add=True)
                pltpu.async_copy(
                    table_sh.at[src_v.at[jnp.minimum(j + nbuf, last)]],
                    bufs[b], sems[b])
            return carry

        lax.fori_loop(0, rows_per_tile // nbuf, body, 0)
        for b in range(nbuf):
            pltpu.make_async_copy(
                table_sh.at[src_v.at[last]], bufs[b], sems[b]).wait()
        plsc.subcore_barrier()

        pltpu.async_copy(acc_sh.at[sl], out_hbm.at[c, sl], sems[0]).wait()

    return segsum


def _proj_body(x_ref, w_ref, b_ref, o_ref):
    o_ref[...] = jnp.dot(x_ref[...], w_ref[...],
                         preferred_element_type=jnp.float32) + b_ref[...]


def _proj_aug_body(x_ref, w_ref, o_ref):
    y = jnp.dot(x_ref[...], w_ref[...], preferred_element_type=jnp.float32)
    ones = jnp.ones((y.shape[0], 1), jnp.float32)
    pad = jnp.zeros((y.shape[0], _W1 - 65), jnp.float32)
    o_ref[...] = jnp.concatenate([y, ones, pad], axis=1)


def _tc2_body(p_ref, r1_ref, wl_ref, h_ref, y2_ref):
    ssum = p_ref[0] + p_ref[1]
    agg = ssum[:, :64]
    deg = ssum[:, 64:65]
    recip = 1.0 / jnp.maximum(deg, 1.0)
    h = jnp.maximum(agg * recip + r1_ref[...], 0.0)
    h_ref[...] = h
    y2_ref[...] = jnp.dot(h, wl_ref[...], preferred_element_type=jnp.float32)


def _tc3_body(p2_ref, dg_ref, r2_ref, out_ref):
    ssum = p2_ref[0] + p2_ref[1]
    deg = dg_ref[0, :, 0:1] + dg_ref[1, :, 0:1]
    recip = 1.0 / jnp.maximum(deg, 1.0)
    out_ref[...] = ssum * recip + r2_ref[...]


def kernel(x, edge_index, W1l, b1, W1r, W2l, b2, W2r):
    n, d = x.shape
    h = W1l.shape[1]
    e = edge_index.shape[1]

    rows_per_tile = -(-e // (_NC * _NS * _CHUNK * 8)) * 8
    chunks = rows_per_tile * _NC * _NS
    e_pad = chunks * _CHUNK
    src = edge_index[0].astype(jnp.int32)
    dst = edge_index[1].astype(jnp.int32)
    src_p = jnp.concatenate([src, jnp.zeros((e_pad - e,), jnp.int32)])
    dst_p = jnp.concatenate([dst, jnp.full((e_pad - e,), n, jnp.int32)])
    src2d = src_p.reshape(chunks, _CHUNK)
    dst2d = dst_p.reshape(chunks, _CHUNK)

    acc_rows = -(-(n + 1) // (_NS * 8)) * (_NS * 8)
    zeros72 = jnp.zeros((acc_rows, _W1), jnp.float32)

    b1r = b1.reshape(1, h)
    b2r = b2.reshape(1, h)
    zb = jnp.zeros((1, h), jnp.float32)

    xblk = 1000 if n % 1000 == 0 else n
    xgrid = n // xblk
    blk = acc_rows // _NS
    ngrid = _NS

    def _proj(inp, w, b, rows, iblk, igrid, wdim):
        return pl.pallas_call(
            _proj_body,
            grid=(igrid,),
            in_specs=[
                pl.BlockSpec((iblk, wdim), lambda i: (i, 0)),
                pl.BlockSpec((wdim, h), lambda i: (0, 0)),
                pl.BlockSpec((1, h), lambda i: (0, 0)),
            ],
            out_specs=pl.BlockSpec((iblk, h), lambda i: (i, 0)),
            out_shape=jax.ShapeDtypeStruct((rows, h), jnp.float32),
        )(inp, w, b)

    y1aug = pl.pallas_call(
        _proj_aug_body,
        grid=(xgrid,),
        in_specs=[
            pl.BlockSpec((xblk, d), lambda i: (i, 0)),
            pl.BlockSpec((d, h), lambda i: (0, 0)),
        ],
        out_specs=pl.BlockSpec((xblk, _W1), lambda i: (i, 0)),
        out_shape=jax.ShapeDtypeStruct((acc_rows, _W1), jnp.float32),
    )(x, W1l)
    r1 = _proj(x, W1r, b1r, acc_rows, xblk, xgrid, d)

    p1 = _make_segsum(acc_rows, _W1, rows_per_tile, 2)(
        y1aug, src2d, dst2d, zeros72)
    dgc = p1[:, :, 64:64 + _DW]

    harr, y2 = pl.pallas_call(
        _tc2_body,
        grid=(ngrid,),
        in_specs=[
            pl.BlockSpec((_NC, blk, _W1), lambda i: (0, i, 0)),
            pl.BlockSpec((blk, h), lambda i: (i, 0)),
            pl.BlockSpec((h, h), lambda i: (0, 0)),
        ],
        out_specs=[
            pl.BlockSpec((blk, h), lambda i: (i, 0)),
            pl.BlockSpec((blk, h), lambda i: (i, 0)),
        ],
        out_shape=[
            jax.ShapeDtypeStruct((acc_rows, h), jnp.float32),
            jax.ShapeDtypeStruct((acc_rows, h), jnp.float32),
        ],
    )(p1, r1, W2l)

    p2 = _make_segsum(acc_rows, h, rows_per_tile, 2)(
        y2, src2d, dst2d, zeros72)
    r2 = _proj(harr, W2r, b2r, acc_rows, blk, ngrid, h)

    oblk = 1000 if n % 1000 == 0 else n
    out = pl.pallas_call(
        _tc3_body,
        grid=(n // oblk,),
        in_specs=[
            pl.BlockSpec((_NC, oblk, h), lambda i: (0, i, 0)),
            pl.BlockSpec((_NC, oblk, _DW), lambda i: (0, i, 0)),
            pl.BlockSpec((oblk, h), lambda i: (i, 0)),
        ],
        out_specs=pl.BlockSpec((oblk, h), lambda i: (i, 0)),
        out_shape=jax.ShapeDtypeStruct((n, h), jnp.float32),
    )(p2, dgc, r2)

    return out


# speedup vs baseline: 1.0695x; 1.0073x over previous
import functools

import jax
import jax.numpy as jnp
from jax import lax
from jax.experimental import pallas as pl
from jax.experimental.pallas import tpu as pltpu
from jax.experimental.pallas import tpu_sc as plsc

_NC = 2
_NS = 16
_CHUNK = 128
_W1 = 72
_DW = 8


def _make_segsum(acc_rows, w, rows_per_tile, extra, nbuf):
    mesh = plsc.VectorSubcoreMesh(core_axis_name="c", subcore_axis_name="s")
    zrows = acc_rows // _NS
    ngrp = rows_per_tile // nbuf
    tail = rows_per_tile % nbuf

    @functools.partial(
        pl.kernel,
        out_type=jax.ShapeDtypeStruct((_NC, acc_rows, w), jnp.float32),
        mesh=mesh,
        scratch_types=[
            pltpu.VMEM((rows_per_tile, _CHUNK), jnp.int32),
            pltpu.VMEM((rows_per_tile, _CHUNK), jnp.int32),
            pltpu.VMEM((1, _CHUNK), jnp.int32),
            pltpu.VMEM((1, _CHUNK), jnp.int32),
            [pltpu.VMEM((_CHUNK, w), jnp.float32) for _ in range(nbuf)],
            [pltpu.SemaphoreType.DMA for _ in range(nbuf)],
            pltpu.VMEM_SHARED((acc_rows, w), jnp.float32),
            pltpu.VMEM_SHARED((acc_rows, w), jnp.float32),
        ],
        compiler_params=pltpu.CompilerParams(use_tc_tiling_on_sc=False),
    )
    def segsum(table_hbm, src_hbm, dst_hbm, zeros_hbm, out_hbm,
               src_v, dst_v, src_x, dst_x, bufs, sems, table_sh, acc_sh):
        c = lax.axis_index("c")
        s = lax.axis_index("s")
        wid = c * _NS + s
        sl = pl.ds(s * zrows, zrows)

        base = wid * rows_per_tile
        stage = [
            pltpu.async_copy(table_hbm.at[sl], table_sh.at[sl], sems[0]),
            pltpu.async_copy(zeros_hbm.at[sl, pl.ds(0, w)], acc_sh.at[sl],
                             sems[0]),
            pltpu.async_copy(src_hbm.at[pl.ds(base, rows_per_tile)], src_v,
                             sems[0]),
            pltpu.async_copy(dst_hbm.at[pl.ds(base, rows_per_tile)], dst_v,
                             sems[0]),
        ]
        for cp in stage:
            cp.wait()
        if extra:
            @pl.when(wid < extra)
            def _():
                xrow = _NC * _NS * rows_per_tile + wid
                cps = [
                    pltpu.async_copy(src_hbm.at[pl.ds(xrow, 1)], src_x,
                                     sems[0]),
                    pltpu.async_copy(dst_hbm.at[pl.ds(xrow, 1)], dst_x,
                                     sems[0]),
                ]
                for cp in cps:
                    cp.wait()
        plsc.subcore_barrier()

        last = rows_per_tile - 1
        for b in range(nbuf):
            pltpu.async_copy(table_sh.at[src_v.at[min(b, last)]],
                             bufs[b], sems[b])

        def body(g, carry):
            for b in range(nbuf):
                j = g * nbuf + b
                pltpu.make_async_copy(
                    table_sh.at[src_v.at[j]], bufs[b], sems[b]).wait()
                pltpu.sync_copy(bufs[b], acc_sh.at[dst_v.at[j]], add=True)
                pltpu.async_copy(
                    table_sh.at[src_v.at[jnp.minimum(j + nbuf, last)]],
                    bufs[b], sems[b])
            return carry

        lax.fori_loop(0, ngrp, body, 0)
        for t in range(tail):
            j = ngrp * nbuf + t
            pltpu.make_async_copy(
                table_sh.at[src_v.at[j]], bufs[t], sems[t]).wait()
            pltpu.sync_copy(bufs[t], acc_sh.at[dst_v.at[j]], add=True)
        for b in range(tail, nbuf):
            pltpu.make_async_copy(
                table_sh.at[src_v.at[last]], bufs[b], sems[b]).wait()
        if extra:
            @pl.when(wid < extra)
            def _():
                pltpu.async_copy(table_sh.at[src_x.at[0]], bufs[0],
                                 sems[0]).wait()
                pltpu.sync_copy(bufs[0], acc_sh.at[dst_x.at[0]], add=True)
        plsc.subcore_barrier()

        pltpu.async_copy(acc_sh.at[sl], out_hbm.at[c, sl], sems[0]).wait()

    return segsum


def _proj_body(x_ref, w_ref, b_ref, o_ref):
    o_ref[...] = jnp.dot(x_ref[...], w_ref[...],
                         preferred_element_type=jnp.float32) + b_ref[...]


def _proj_aug_body(x_ref, w_ref, o_ref):
    y = jnp.dot(x_ref[...], w_ref[...], preferred_element_type=jnp.float32)
    ones = jnp.ones((y.shape[0], 1), jnp.float32)
    pad = jnp.zeros((y.shape[0], _W1 - 65), jnp.float32)
    o_ref[...] = jnp.concatenate([y, ones, pad], axis=1)


def _tc2_body(p_ref, r1_ref, wl_ref, h_ref, y2_ref):
    ssum = p_ref[0] + p_ref[1]
    agg = ssum[:, :64]
    deg = ssum[:, 64:65]
    recip = 1.0 / jnp.maximum(deg, 1.0)
    h = jnp.maximum(agg * recip + r1_ref[...], 0.0)
    h_ref[...] = h
    y2_ref[...] = jnp.dot(h, wl_ref[...], preferred_element_type=jnp.float32)


def _tc3_body(p2_ref, dg_ref, r2_ref, out_ref):
    ssum = p2_ref[0] + p2_ref[1]
    deg = dg_ref[0, :, 0:1] + dg_ref[1, :, 0:1]
    recip = 1.0 / jnp.maximum(deg, 1.0)
    out_ref[...] = ssum * recip + r2_ref[...]


def kernel(x, edge_index, W1l, b1, W1r, W2l, b2, W2r):
    n, d = x.shape
    h = W1l.shape[1]
    e = edge_index.shape[1]

    chunks = -(-e // _CHUNK)
    e_pad = chunks * _CHUNK
    src = edge_index[0].astype(jnp.int32)
    dst = edge_index[1].astype(jnp.int32)
    if e_pad > e:
        src = jnp.concatenate([src, jnp.zeros((e_pad - e,), jnp.int32)])
        dst = jnp.concatenate([dst, jnp.full((e_pad - e,), n, jnp.int32)])
    src2d = src.reshape(chunks, _CHUNK)
    dst2d = dst.reshape(chunks, _CHUNK)
    rows_per_tile = chunks // (_NC * _NS)
    extra = chunks % (_NC * _NS)

    acc_rows = -(-(n + 1) // (_NS * 8)) * (_NS * 8)
    zeros72 = jnp.zeros((acc_rows, _W1), jnp.float32)

    b1r = b1.reshape(1, h)
    b2r = b2.reshape(1, h)
    zb = jnp.zeros((1, h), jnp.float32)

    xblk = 1000 if n % 1000 == 0 else n
    xgrid = n // xblk
    blk = acc_rows // _NS
    ngrid = _NS

    def _proj(inp, w, b, rows, iblk, igrid, wdim):
        return pl.pallas_call(
            _proj_body,
            grid=(igrid,),
            in_specs=[
                pl.BlockSpec((iblk, wdim), lambda i: (i, 0)),
                pl.BlockSpec((wdim, h), lambda i: (0, 0)),
                pl.BlockSpec((1, h), lambda i: (0, 0)),
            ],
            out_specs=pl.BlockSpec((iblk, h), lambda i: (i, 0)),
            out_shape=jax.ShapeDtypeStruct((rows, h), jnp.float32),
        )(inp, w, b)

    y1aug = pl.pallas_call(
        _proj_aug_body,
        grid=(xgrid,),
        in_specs=[
            pl.BlockSpec((xblk, d), lambda i: (i, 0)),
            pl.BlockSpec((d, h), lambda i: (0, 0)),
        ],
        out_specs=pl.BlockSpec((xblk, _W1), lambda i: (i, 0)),
        out_shape=jax.ShapeDtypeStruct((acc_rows, _W1), jnp.float32),
    )(x, W1l)
    r1 = _proj(x, W1r, b1r, acc_rows, xblk, xgrid, d)

    p1 = _make_segsum(acc_rows, _W1, rows_per_tile, extra, 2)(
        y1aug, src2d, dst2d, zeros72)
    dgc = p1[:, :, 64:64 + _DW]

    harr, y2 = pl.pallas_call(
        _tc2_body,
        grid=(ngrid,),
        in_specs=[
            pl.BlockSpec((_NC, blk, _W1), lambda i: (0, i, 0)),
            pl.BlockSpec((blk, h), lambda i: (i, 0)),
            pl.BlockSpec((h, h), lambda i: (0, 0)),
        ],
        out_specs=[
            pl.BlockSpec((blk, h), lambda i: (i, 0)),
            pl.BlockSpec((blk, h), lambda i: (i, 0)),
        ],
        out_shape=[
            jax.ShapeDtypeStruct((acc_rows, h), jnp.float32),
            jax.ShapeDtypeStruct((acc_rows, h), jnp.float32),
        ],
    )(p1, r1, W2l)

    p2 = _make_segsum(acc_rows, h, rows_per_tile, extra, 2)(
        y2, src2d, dst2d, zeros72)
    r2 = _proj(harr, W2r, b2r, acc_rows, blk, ngrid, h)

    oblk = 1000 if n % 1000 == 0 else n
    out = pl.pallas_call(
        _tc3_body,
        grid=(n // oblk,),
        in_specs=[
            pl.BlockSpec((_NC, oblk, h), lambda i: (0, i, 0)),
            pl.BlockSpec((_NC, oblk, _DW), lambda i: (0, i, 0)),
            pl.BlockSpec((oblk, h), lambda i: (i, 0)),
        ],
        out_specs=pl.BlockSpec((oblk, h), lambda i: (i, 0)),
        out_shape=jax.ShapeDtypeStruct((n, h), jnp.float32),
    )(p2, dgc, r2)

    return out
